# Initial kernel scaffold; baseline (speedup 1.0000x reference)
import jax
import jax.numpy as jnp
from jax.experimental import pallas as pl


def kernel(x, edge_index, batch, W_rel1, b1, W_root1, p1, W_rel2, b2, W_root2, p2, W_rel3, b3, W_root3, p3):
    raise NotImplementedError("write your pallas kernel here")


# speedup vs baseline: 11.0651x; 11.0651x over previous
import functools

import jax
import jax.numpy as jnp
from jax import lax
from jax.experimental import pallas as pl
from jax.experimental.pallas import tpu as pltpu
from jax.experimental.pallas import tpu_sc as plsc

N = 10000
E = 320000
D = 128
NC = 2
NS = 16
NW = NC * NS
EPW = E // NW
CH = 80
NCHUNK = EPW // CH
NPAD = 10240
RPS = NPAD // NS
ZR = 128
RB = 1024
SR = NPAD // D

_f32 = jnp.float32
_i32 = jnp.int32


def _sc_aggregate_body(f_hbm, src_hbm, dst_hbm, out_hbm,
                       src_v, dst_v, rows_v, zbuf, acc, sem):
    c = lax.axis_index("c")
    s = lax.axis_index("s")
    wid = c * NS + s

    zero16 = jnp.zeros((16,), _f32)

    def _zrow(r, carry):
        for j in range(D // 16):
            zbuf[r, pl.ds(j * 16, 16)] = zero16
        return carry

    lax.fori_loop(0, ZR, _zrow, 0)
    for j in range(RPS // ZR):
        pltpu.sync_copy(zbuf, acc.at[pl.ds(s * RPS + j * ZR, ZR)])
    plsc.subcore_barrier()

    base = wid * EPW

    def _chunk(i, carry):
        off = base + i * CH
        pltpu.sync_copy(src_hbm.at[pl.ds(off, CH)], src_v)
        pltpu.sync_copy(dst_hbm.at[pl.ds(off, CH)], dst_v)
        pltpu.async_copy(f_hbm.at[src_v], rows_v, sem).wait()
        pltpu.sync_copy(rows_v, acc.at[dst_v], add=True)
        return carry

    lax.fori_loop(0, NCHUNK, _chunk, 0)
    plsc.subcore_barrier()

    for j in range(RPS // ZR):
        r0 = s * RPS + j * ZR
        pltpu.sync_copy(acc.at[pl.ds(r0, ZR)], out_hbm.at[c, pl.ds(r0, ZR)])


@functools.cache
def _sc_aggregate():
    mesh = plsc.VectorSubcoreMesh(core_axis_name="c", subcore_axis_name="s",
                                  num_cores=NC, num_subcores=NS)
    return pl.kernel(
        _sc_aggregate_body,
        mesh=mesh,
        out_type=jax.ShapeDtypeStruct((NC, NPAD, D), _f32),
        scratch_types=[
            pltpu.VMEM((CH,), _i32),
            pltpu.VMEM((CH,), _i32),
            pltpu.VMEM((CH, D), _f32),
            pltpu.VMEM((ZR, D), _f32),
            pltpu.VMEM_SHARED((NPAD, D), _f32),
            pltpu.SemaphoreType.DMA,
        ],
    )


def _monotone_u32(x):
    u = lax.bitcast_convert_type(x, jnp.uint32)
    return jnp.where((u >> 31) != 0, ~u, u | jnp.uint32(0x80000000))


def _matmul_body(aggr_ref, x_ref, wrel_ref, b_ref, wroot_ref, p_ref,
                 h_ref, sc_ref):
    a = aggr_ref[0] + aggr_ref[1]
    h = jnp.dot(a, wrel_ref[...], preferred_element_type=_f32)
    h = h + b_ref[...] + jnp.dot(x_ref[...], wroot_ref[...],
                                 preferred_element_type=_f32)
    h = jnp.maximum(h, 0.0)
    p = p_ref[...]
    inv_norm = lax.rsqrt(jnp.sum(p * p))
    h_ref[...] = h
    sc_ref[...] = jnp.dot(h, p, preferred_element_type=_f32) * inv_norm


def _tc_matmul(aggr, xprev, wrel, b, wroot, p):
    grid = NPAD // RB
    return pl.pallas_call(
        _matmul_body,
        grid=(grid,),
        in_specs=[
            pl.BlockSpec((NC, RB, D), lambda i: (0, i, 0)),
            pl.BlockSpec((RB, D), lambda i: (i, 0)),
            pl.BlockSpec((D, D), lambda i: (0, 0)),
            pl.BlockSpec((1, D), lambda i: (0, 0)),
            pl.BlockSpec((D, D), lambda i: (0, 0)),
            pl.BlockSpec((D, 1), lambda i: (0, 0)),
        ],
        out_specs=[
            pl.BlockSpec((RB, D), lambda i: (i, 0)),
            pl.BlockSpec((RB, 1), lambda i: (i, 0)),
        ],
        out_shape=[jax.ShapeDtypeStruct((NPAD, D), _f32),
                   jax.ShapeDtypeStruct((NPAD, 1), _f32)],
    )(aggr, xprev, wrel, b.reshape(1, D), wroot, p.reshape(D, 1))


def _thresh_body(sc_ref, alive_ref, alive_out_ref, *, k):
    sm = jnp.where(alive_ref[...] > 0.0, sc_ref[...], _f32(-3e38))
    u = _monotone_u32(sm)

    def body(i, pfx):
        cand = pfx | (jnp.uint32(1) << (31 - i))
        cnt = jnp.sum((u >= cand).astype(_i32))
        return jnp.where(cnt >= k, cand, pfx)

    thr = lax.fori_loop(0, 32, body, jnp.uint32(0))
    alive_out_ref[...] = (u >= thr).astype(_f32)


def _tc_threshold(sc_r, alive_prev_r, k):
    return pl.pallas_call(
        functools.partial(_thresh_body, k=k),
        out_shape=jax.ShapeDtypeStruct((SR, D), _f32),
    )(sc_r, alive_prev_r)


def _apply_body(h_ref, sc_ref, alive_ref, f_ref):
    f_ref[...] = h_ref[...] * jnp.tanh(sc_ref[...]) * alive_ref[...]


def _tc_apply(h, sc, alive):
    grid = NPAD // RB
    return pl.pallas_call(
        _apply_body,
        grid=(grid,),
        in_specs=[
            pl.BlockSpec((RB, D), lambda i: (i, 0)),
            pl.BlockSpec((RB, 1), lambda i: (i, 0)),
            pl.BlockSpec((RB, 1), lambda i: (i, 0)),
        ],
        out_specs=pl.BlockSpec((RB, D), lambda i: (i, 0)),
        out_shape=jax.ShapeDtypeStruct((NPAD, D), _f32),
    )(h, sc, alive)


def _final_body(h_ref, sc_ref, scr_ref, alive_ref, out_ref):
    sm = jnp.where(alive_ref[...] > 0.0, scr_ref[...], _f32(-3e38))
    fiota = (lax.broadcasted_iota(_i32, (SR, D), 0) * D
             + lax.broadcasted_iota(_i32, (SR, D), 1))
    rows = []
    svals = []
    for _ in range(10):
        m = jnp.max(sm)
        idx = jnp.min(jnp.where(sm == m, fiota, NPAD))
        rows.append(h_ref[pl.ds(idx, 1), :])
        svals.append(sc_ref[pl.ds(idx, 1), :])
        sm = jnp.where(fiota == idx, _f32(-3e38), sm)
    v = jnp.concatenate(rows, axis=0) * jnp.tanh(jnp.concatenate(svals, axis=0))
    mx = jnp.max(v, axis=1, keepdims=True)
    v = v - mx
    out_ref[...] = v - jnp.log(jnp.sum(jnp.exp(v), axis=1, keepdims=True))


def _tc_final(h, sc, sc_r, alive_prev_r):
    return pl.pallas_call(
        _final_body,
        out_shape=jax.ShapeDtypeStruct((10, D), _f32),
    )(h, sc, sc_r, alive_prev_r)


def kernel(x, edge_index, batch, W_rel1, b1, W_root1, p1,
           W_rel2, b2, W_root2, p2, W_rel3, b3, W_root3, p3):
    src = edge_index[0].astype(_i32)
    dst = edge_index[1].astype(_i32)
    xp = jnp.pad(x, ((0, NPAD - N), (0, 0)))
    alive0_r = (jnp.arange(NPAD, dtype=_i32).reshape(SR, D) < N).astype(_f32)

    sc_agg = _sc_aggregate()

    agg1 = sc_agg(xp, src, dst)
    h1, s1 = _tc_matmul(agg1, xp, W_rel1, b1, W_root1, p1)
    alive1_r = _tc_threshold(s1.reshape(SR, D), alive0_r, 8000)
    f1 = _tc_apply(h1, s1, alive1_r.reshape(NPAD, 1))

    agg2 = sc_agg(f1, src, dst)
    h2, s2 = _tc_matmul(agg2, f1, W_rel2, b2, W_root2, p2)
    alive2_r = _tc_threshold(s2.reshape(SR, D), alive1_r, 6400)
    f2 = _tc_apply(h2, s2, alive2_r.reshape(NPAD, 1))

    agg3 = sc_agg(f2, src, dst)
    h3, s3 = _tc_matmul(agg3, f2, W_rel3, b3, W_root3, p3)
    return _tc_final(h3, s3, s3.reshape(SR, D), alive2_r)


# speedup vs baseline: 19.4005x; 1.7533x over previous
import functools

import jax
import jax.numpy as jnp
from jax import lax
from jax.experimental import pallas as pl
from jax.experimental.pallas import tpu as pltpu
from jax.experimental.pallas import tpu_sc as plsc

N = 10000
E = 320000
D = 128
NC = 2
NS = 16
NW = NC * NS
CH = 80
NCHUNK = 125
EPW = NCHUNK * CH
NPAD = 10240
RPS = NPAD // NS
RB = 1024
SR = NPAD // D

_f32 = jnp.float32
_i32 = jnp.int32


def _sc_aggregate_body(f_hbm, src_hbm, dst_hbm, out_hbm,
                       srcs_v, dsts_v, rows_v, acc, sem):
    c = lax.axis_index("c")
    s = lax.axis_index("s")
    wid = c * NS + s

    pltpu.sync_copy(src_hbm.at[wid], srcs_v)
    pltpu.sync_copy(dst_hbm.at[wid], dsts_v)

    def _src_at(i):
        return srcs_v.at[pl.ds(i * CH, CH)]

    zero16 = jnp.zeros((16,), _f32)

    def _zrow(r, carry):
        for j in range(D // 16):
            rows_v[0, r, pl.ds(j * 16, 16)] = zero16
        return carry

    lax.fori_loop(0, CH, _zrow, 0)
    for j in range(RPS // CH):
        pltpu.sync_copy(rows_v.at[0], acc.at[pl.ds(s * RPS + j * CH, CH)])
    plsc.subcore_barrier()

    pltpu.async_copy(f_hbm.at[_src_at(0)], rows_v.at[0], sem)

    def _pair(i2, carry):
        i0 = 2 * i2
        for b in range(2):
            i = i0 + b
            pltpu.make_async_copy(f_hbm.at[_src_at(i)], rows_v.at[b],
                                  sem).wait()
            pltpu.async_copy(f_hbm.at[_src_at(i + 1)], rows_v.at[1 - b],
                             sem)
            pltpu.sync_copy(rows_v.at[b], acc.at[dsts_v.at[i]], add=True)
        return carry

    lax.fori_loop(0, NCHUNK // 2, _pair, 0)
    ilast = NCHUNK - 1
    pltpu.make_async_copy(f_hbm.at[_src_at(ilast)], rows_v.at[0],
                          sem).wait()
    pltpu.sync_copy(rows_v.at[0], acc.at[dsts_v.at[ilast]], add=True)
    plsc.subcore_barrier()

    for j in range(RPS // CH):
        r0 = s * RPS + j * CH
        pltpu.sync_copy(acc.at[pl.ds(r0, CH)], out_hbm.at[c, pl.ds(r0, CH)])


@functools.cache
def _sc_aggregate():
    mesh = plsc.VectorSubcoreMesh(core_axis_name="c", subcore_axis_name="s",
                                  num_cores=NC, num_subcores=NS)
    return pl.kernel(
        _sc_aggregate_body,
        mesh=mesh,
        out_type=jax.ShapeDtypeStruct((NC, NPAD, D), _f32),
        scratch_types=[
            pltpu.VMEM((EPW,), _i32),
            pltpu.VMEM((NCHUNK, CH), _i32),
            pltpu.VMEM((2, CH, D), _f32),
            pltpu.VMEM_SHARED((NPAD, D), _f32),
            pltpu.SemaphoreType.DMA,
        ],
    )


def _monotone_u32(x):
    u = lax.bitcast_convert_type(x, jnp.uint32)
    return jnp.where((u >> 31) != 0, ~u, u | jnp.uint32(0x80000000))


def _matmul_body(aggr_ref, x_ref, wrel_ref, b_ref, wroot_ref, p_ref,
                 h_ref, sc_ref):
    a = aggr_ref[0] + aggr_ref[1]
    h = jnp.dot(a, wrel_ref[...], preferred_element_type=_f32)
    h = h + b_ref[...] + jnp.dot(x_ref[...], wroot_ref[...],
                                 preferred_element_type=_f32)
    h = jnp.maximum(h, 0.0)
    p = p_ref[...]
    inv_norm = lax.rsqrt(jnp.sum(p * p))
    h_ref[...] = h
    sc_ref[...] = jnp.dot(h, p, preferred_element_type=_f32) * inv_norm


def _tc_matmul(aggr, xprev, wrel, b, wroot, p):
    grid = NPAD // RB
    return pl.pallas_call(
        _matmul_body,
        grid=(grid,),
        in_specs=[
            pl.BlockSpec((NC, RB, D), lambda i: (0, i, 0)),
            pl.BlockSpec((RB, D), lambda i: (i, 0)),
            pl.BlockSpec((D, D), lambda i: (0, 0)),
            pl.BlockSpec((1, D), lambda i: (0, 0)),
            pl.BlockSpec((D, D), lambda i: (0, 0)),
            pl.BlockSpec((D, 1), lambda i: (0, 0)),
        ],
        out_specs=[
            pl.BlockSpec((RB, D), lambda i: (i, 0)),
            pl.BlockSpec((RB, 1), lambda i: (i, 0)),
        ],
        out_shape=[jax.ShapeDtypeStruct((NPAD, D), _f32),
                   jax.ShapeDtypeStruct((NPAD, 1), _f32)],
    )(aggr, xprev, wrel, b.reshape(1, D), wroot, p.reshape(D, 1))


def _thresh_body(sc_ref, alive_ref, alive_out_ref, *, k):
    sm = jnp.where(alive_ref[...] > 0.0, sc_ref[...], _f32(-3e38))
    u = _monotone_u32(sm)

    def body(i, pfx):
        cand = pfx | (jnp.uint32(1) << (31 - i))
        cnt = jnp.sum((u >= cand).astype(_i32))
        return jnp.where(cnt >= k, cand, pfx)

    thr = lax.fori_loop(0, 32, body, jnp.uint32(0))
    alive_out_ref[...] = (u >= thr).astype(_f32)


def _tc_threshold(sc_r, alive_prev_r, k):
    return pl.pallas_call(
        functools.partial(_thresh_body, k=k),
        out_shape=jax.ShapeDtypeStruct((SR, D), _f32),
    )(sc_r, alive_prev_r)


def _apply_body(h_ref, sc_ref, alive_ref, f_ref):
    f_ref[...] = h_ref[...] * jnp.tanh(sc_ref[...]) * alive_ref[...]


def _tc_apply(h, sc, alive):
    grid = NPAD // RB
    return pl.pallas_call(
        _apply_body,
        grid=(grid,),
        in_specs=[
            pl.BlockSpec((RB, D), lambda i: (i, 0)),
            pl.BlockSpec((RB, 1), lambda i: (i, 0)),
            pl.BlockSpec((RB, 1), lambda i: (i, 0)),
        ],
        out_specs=pl.BlockSpec((RB, D), lambda i: (i, 0)),
        out_shape=jax.ShapeDtypeStruct((NPAD, D), _f32),
    )(h, sc, alive)


def _final_body(h_ref, sc_ref, scr_ref, alive_ref, out_ref):
    sm = jnp.where(alive_ref[...] > 0.0, scr_ref[...], _f32(-3e38))
    fiota = (lax.broadcasted_iota(_i32, (SR, D), 0) * D
             + lax.broadcasted_iota(_i32, (SR, D), 1))
    rows = []
    svals = []
    for _ in range(10):
        m = jnp.max(sm)
        idx = jnp.min(jnp.where(sm == m, fiota, NPAD))
        rows.append(h_ref[pl.ds(idx, 1), :])
        svals.append(sc_ref[pl.ds(idx, 1), :])
        sm = jnp.where(fiota == idx, _f32(-3e38), sm)
    v = jnp.concatenate(rows, axis=0) * jnp.tanh(jnp.concatenate(svals, axis=0))
    mx = jnp.max(v, axis=1, keepdims=True)
    v = v - mx
    out_ref[...] = v - jnp.log(jnp.sum(jnp.exp(v), axis=1, keepdims=True))


def _tc_final(h, sc, sc_r, alive_prev_r):
    return pl.pallas_call(
        _final_body,
        out_shape=jax.ShapeDtypeStruct((10, D), _f32),
    )(h, sc, sc_r, alive_prev_r)


def kernel(x, edge_index, batch, W_rel1, b1, W_root1, p1,
           W_rel2, b2, W_root2, p2, W_rel3, b3, W_root3, p3):
    src = edge_index[0].astype(_i32).reshape(NW, EPW)
    dst = edge_index[1].astype(_i32).reshape(NW, NCHUNK, CH)
    xp = jnp.pad(x, ((0, NPAD - N), (0, 0)))
    alive0_r = (jnp.arange(NPAD, dtype=_i32).reshape(SR, D) < N).astype(_f32)

    sc_agg = _sc_aggregate()

    agg1 = sc_agg(xp, src, dst)
    h1, s1 = _tc_matmul(agg1, xp, W_rel1, b1, W_root1, p1)
    alive1_r = _tc_threshold(s1.reshape(SR, D), alive0_r, 8000)
    f1 = _tc_apply(h1, s1, alive1_r.reshape(NPAD, 1))

    agg2 = sc_agg(f1, src, dst)
    h2, s2 = _tc_matmul(agg2, f1, W_rel2, b2, W_root2, p2)
    alive2_r = _tc_threshold(s2.reshape(SR, D), alive1_r, 6400)
    f2 = _tc_apply(h2, s2, alive2_r.reshape(NPAD, 1))

    agg3 = sc_agg(f2, src, dst)
    h3, s3 = _tc_matmul(agg3, f2, W_rel3, b3, W_root3, p3)
    return _tc_final(h3, s3, s3.reshape(SR, D), alive2_r)


# speedup vs baseline: 19.6475x; 1.0127x over previous
import functools

import jax
import jax.numpy as jnp
from jax import lax
from jax.experimental import pallas as pl
from jax.experimental.pallas import tpu as pltpu
from jax.experimental.pallas import tpu_sc as plsc

N = 10000
E = 320000
D = 128
NC = 2
NS = 16
NW = NC * NS
CH = 80
NCHUNK = 125
EPW = NCHUNK * CH
NPAD = 10240
RPS = NPAD // NS
RB = 1024
SR = NPAD // D

_f32 = jnp.float32
_i32 = jnp.int32


def _sc_aggregate_body(f_hbm, src_hbm, dst_hbm, out_hbm,
                       srcs_v, dsts_v, rows_v, acc, sem, sem_s):
    c = lax.axis_index("c")
    s = lax.axis_index("s")
    wid = c * NS + s

    pltpu.sync_copy(src_hbm.at[wid], srcs_v)
    pltpu.sync_copy(dst_hbm.at[wid], dsts_v)

    def _src_at(i):
        return srcs_v.at[pl.ds(i * CH, CH)]

    zero16 = jnp.zeros((16,), _f32)

    def _zrow(r, carry):
        for j in range(D // 16):
            rows_v[0, r, pl.ds(j * 16, 16)] = zero16
        return carry

    lax.fori_loop(0, CH, _zrow, 0)
    for j in range(RPS // CH):
        pltpu.sync_copy(rows_v.at[0], acc.at[pl.ds(s * RPS + j * CH, CH)])
    plsc.subcore_barrier()

    pltpu.async_copy(f_hbm.at[_src_at(0)], rows_v.at[0], sem)

    def _step(i, b):
        pltpu.make_async_copy(f_hbm.at[_src_at(i)], rows_v.at[b], sem).wait()
        pltpu.async_copy(rows_v.at[b], acc.at[dsts_v.at[i]], sem_s, add=True)

    def _drain_one(i, b):
        pltpu.make_async_copy(rows_v.at[b], acc.at[dsts_v.at[i]],
                              sem_s).wait()

    _step(0, 0)
    pltpu.async_copy(f_hbm.at[_src_at(1)], rows_v.at[1], sem)

    def _pair(i2, carry):
        for b in range(2):
            i = 2 * i2 + 1 + b
            _step(i, 1 - b)
            _drain_one(i - 1, b)
            pltpu.async_copy(f_hbm.at[_src_at(i + 1)], rows_v.at[b], sem)
        return carry

    lax.fori_loop(0, (NCHUNK - 3) // 2, _pair, 0)
    _step(NCHUNK - 2, 1)
    _drain_one(NCHUNK - 3, 0)
    pltpu.async_copy(f_hbm.at[_src_at(NCHUNK - 1)], rows_v.at[0], sem)
    _step(NCHUNK - 1, 0)
    _drain_one(NCHUNK - 2, 1)
    _drain_one(NCHUNK - 1, 0)
    plsc.subcore_barrier()

    for j in range(RPS // CH):
        r0 = s * RPS + j * CH
        pltpu.sync_copy(acc.at[pl.ds(r0, CH)], out_hbm.at[c, pl.ds(r0, CH)])


@functools.cache
def _sc_aggregate():
    mesh = plsc.VectorSubcoreMesh(core_axis_name="c", subcore_axis_name="s",
                                  num_cores=NC, num_subcores=NS)
    return pl.kernel(
        _sc_aggregate_body,
        mesh=mesh,
        out_type=jax.ShapeDtypeStruct((NC, NPAD, D), _f32),
        scratch_types=[
            pltpu.VMEM((EPW,), _i32),
            pltpu.VMEM((NCHUNK, CH), _i32),
            pltpu.VMEM((2, CH, D), _f32),
            pltpu.VMEM_SHARED((NPAD, D), _f32),
            pltpu.SemaphoreType.DMA,
            pltpu.SemaphoreType.DMA,
        ],
    )


def _monotone_u32(x):
    u = lax.bitcast_convert_type(x, jnp.uint32)
    return jnp.where((u >> 31) != 0, ~u, u | jnp.uint32(0x80000000))


def _matmul_body(aggr_ref, x_ref, wrel_ref, b_ref, wroot_ref, p_ref,
                 h_ref, sc_ref):
    a = aggr_ref[0] + aggr_ref[1]
    h = jnp.dot(a, wrel_ref[...], preferred_element_type=_f32)
    h = h + b_ref[...] + jnp.dot(x_ref[...], wroot_ref[...],
                                 preferred_element_type=_f32)
    h = jnp.maximum(h, 0.0)
    p = p_ref[...]
    inv_norm = lax.rsqrt(jnp.sum(p * p))
    h_ref[...] = h
    sc_ref[...] = jnp.dot(h, p, preferred_element_type=_f32) * inv_norm


def _tc_matmul(aggr, xprev, wrel, b, wroot, p):
    grid = NPAD // RB
    return pl.pallas_call(
        _matmul_body,
        grid=(grid,),
        in_specs=[
            pl.BlockSpec((NC, RB, D), lambda i: (0, i, 0)),
            pl.BlockSpec((RB, D), lambda i: (i, 0)),
            pl.BlockSpec((D, D), lambda i: (0, 0)),
            pl.BlockSpec((1, D), lambda i: (0, 0)),
            pl.BlockSpec((D, D), lambda i: (0, 0)),
            pl.BlockSpec((D, 1), lambda i: (0, 0)),
        ],
        out_specs=[
            pl.BlockSpec((RB, D), lambda i: (i, 0)),
            pl.BlockSpec((RB, 1), lambda i: (i, 0)),
        ],
        out_shape=[jax.ShapeDtypeStruct((NPAD, D), _f32),
                   jax.ShapeDtypeStruct((NPAD, 1), _f32)],
    )(aggr, xprev, wrel, b.reshape(1, D), wroot, p.reshape(D, 1))


def _pool_body(h_ref, sc_ref, scr_ref, alive_ref, alivec_ref,
               f_ref, alive_out_ref, *, k):
    sm = jnp.where(alive_ref[...] > 0.0, scr_ref[...], _f32(-3e38))
    u = _monotone_u32(sm)

    def body(i, pfx):
        cand = pfx | (jnp.uint32(1) << (31 - i))
        cnt = jnp.sum((u >= cand).astype(_i32))
        return jnp.where(cnt >= k, cand, pfx)

    thr = lax.fori_loop(0, 32, body, jnp.uint32(0))
    alive_out_ref[...] = (u >= thr).astype(_f32)
    sc = sc_ref[...]
    alive_col = jnp.where(
        (_monotone_u32(sc) >= thr) & (alivec_ref[...] > 0.0), 1.0, 0.0)
    f_ref[...] = h_ref[...] * jnp.tanh(sc) * alive_col


def _tc_pool(h, sc, sc_r, alive_prev_r, alive_prev_col, k):
    return pl.pallas_call(
        functools.partial(_pool_body, k=k),
        out_shape=[jax.ShapeDtypeStruct((NPAD, D), _f32),
                   jax.ShapeDtypeStruct((SR, D), _f32)],
    )(h, sc, sc_r, alive_prev_r, alive_prev_col)


def _final_body(h_ref, sc_ref, scr_ref, alive_ref, out_ref):
    sm = jnp.where(alive_ref[...] > 0.0, scr_ref[...], _f32(-3e38))
    fiota = (lax.broadcasted_iota(_i32, (SR, D), 0) * D
             + lax.broadcasted_iota(_i32, (SR, D), 1))
    rows = []
    svals = []
    for _ in range(10):
        m = jnp.max(sm)
        idx = jnp.min(jnp.where(sm == m, fiota, NPAD))
        rows.append(h_ref[pl.ds(idx, 1), :])
        svals.append(sc_ref[pl.ds(idx, 1), :])
        sm = jnp.where(fiota == idx, _f32(-3e38), sm)
    v = jnp.concatenate(rows, axis=0) * jnp.tanh(jnp.concatenate(svals, axis=0))
    mx = jnp.max(v, axis=1, keepdims=True)
    v = v - mx
    out_ref[...] = v - jnp.log(jnp.sum(jnp.exp(v), axis=1, keepdims=True))


def _tc_final(h, sc, sc_r, alive_prev_r):
    return pl.pallas_call(
        _final_body,
        out_shape=jax.ShapeDtypeStruct((10, D), _f32),
    )(h, sc, sc_r, alive_prev_r)


def kernel(x, edge_index, batch, W_rel1, b1, W_root1, p1,
           W_rel2, b2, W_root2, p2, W_rel3, b3, W_root3, p3):
    src = edge_index[0].astype(_i32).reshape(NW, EPW)
    dst = edge_index[1].astype(_i32).reshape(NW, NCHUNK, CH)
    xp = jnp.pad(x, ((0, NPAD - N), (0, 0)))
    alive0_r = (jnp.arange(NPAD, dtype=_i32).reshape(SR, D) < N).astype(_f32)
    alive0_c = alive0_r.reshape(NPAD, 1)

    sc_agg = _sc_aggregate()

    agg1 = sc_agg(xp, src, dst)
    h1, s1 = _tc_matmul(agg1, xp, W_rel1, b1, W_root1, p1)
    f1, alive1_r = _tc_pool(h1, s1, s1.reshape(SR, D), alive0_r, alive0_c,
                            8000)

    agg2 = sc_agg(f1, src, dst)
    h2, s2 = _tc_matmul(agg2, f1, W_rel2, b2, W_root2, p2)
    f2, alive2_r = _tc_pool(h2, s2, s2.reshape(SR, D), alive1_r,
                            alive1_r.reshape(NPAD, 1), 6400)

    agg3 = sc_agg(f2, src, dst)
    h3, s3 = _tc_matmul(agg3, f2, W_rel3, b3, W_root3, p3)
    return _tc_final(h3, s3, s3.reshape(SR, D), alive2_r)


# speedup vs baseline: 19.7012x; 1.0027x over previous
import functools

import jax
import jax.numpy as jnp
from jax import lax
from jax.experimental import pallas as pl
from jax.experimental.pallas import tpu as pltpu
from jax.experimental.pallas import tpu_sc as plsc

N = 10000
E = 320000
D = 128
NC = 2
NS = 16
NW = NC * NS
CH = 80
NCHUNK = 125
EPW = NCHUNK * CH
NPAD = 10240
RPS = NPAD // NS
RB = 1024
SR = NPAD // D

_f32 = jnp.float32
_i32 = jnp.int32


def _sc_aggregate_body(f_hbm, src_hbm, dst_hbm, out_hbm,
                       srcs_v, dsts_v, rows_v, acc, sem):
    c = lax.axis_index("c")
    s = lax.axis_index("s")
    wid = c * NS + s

    pltpu.sync_copy(src_hbm.at[wid], srcs_v)
    pltpu.sync_copy(dst_hbm.at[wid], dsts_v)

    def _src_at(i):
        return srcs_v.at[pl.ds(i * CH, CH)]

    zero16 = jnp.zeros((16,), _f32)

    def _zrow(r, carry):
        for j in range(D // 16):
            rows_v[0, r, pl.ds(j * 16, 16)] = zero16
        return carry

    lax.fori_loop(0, CH, _zrow, 0)
    for j in range(RPS // CH):
        pltpu.sync_copy(rows_v.at[0], acc.at[pl.ds(s * RPS + j * CH, CH)])
    plsc.subcore_barrier()

    pltpu.async_copy(f_hbm.at[_src_at(0)], rows_v.at[0], sem)

    def _pair(i2, carry):
        i0 = 2 * i2
        for b in range(2):
            i = i0 + b
            pltpu.make_async_copy(f_hbm.at[_src_at(i)], rows_v.at[b],
                                  sem).wait()
            pltpu.async_copy(f_hbm.at[_src_at(i + 1)], rows_v.at[1 - b],
                             sem)
            pltpu.sync_copy(rows_v.at[b], acc.at[dsts_v.at[i]], add=True)
        return carry

    lax.fori_loop(0, NCHUNK // 2, _pair, 0)
    ilast = NCHUNK - 1
    pltpu.make_async_copy(f_hbm.at[_src_at(ilast)], rows_v.at[0],
                          sem).wait()
    pltpu.sync_copy(rows_v.at[0], acc.at[dsts_v.at[ilast]], add=True)
    plsc.subcore_barrier()

    for j in range(RPS // CH):
        r0 = s * RPS + j * CH
        pltpu.sync_copy(acc.at[pl.ds(r0, CH)], out_hbm.at[c, pl.ds(r0, CH)])


@functools.cache
def _sc_aggregate():
    mesh = plsc.VectorSubcoreMesh(core_axis_name="c", subcore_axis_name="s",
                                  num_cores=NC, num_subcores=NS)
    return pl.kernel(
        _sc_aggregate_body,
        mesh=mesh,
        out_type=jax.ShapeDtypeStruct((NC, NPAD, D), _f32),
        scratch_types=[
            pltpu.VMEM((EPW,), _i32),
            pltpu.VMEM((NCHUNK, CH), _i32),
            pltpu.VMEM((2, CH, D), _f32),
            pltpu.VMEM_SHARED((NPAD, D), _f32),
            pltpu.SemaphoreType.DMA,
        ],
    )


def _monotone_u32(x):
    u = lax.bitcast_convert_type(x, jnp.uint32)
    return jnp.where((u >> 31) != 0, ~u, u | jnp.uint32(0x80000000))


def _matmul_body(aggr_ref, x_ref, wrel_ref, b_ref, wroot_ref, p_ref,
                 h_ref, sc_ref):
    a = aggr_ref[0] + aggr_ref[1]
    h = jnp.dot(a, wrel_ref[...], preferred_element_type=_f32)
    h = h + b_ref[...] + jnp.dot(x_ref[...], wroot_ref[...],
                                 preferred_element_type=_f32)
    h = jnp.maximum(h, 0.0)
    p = p_ref[...]
    inv_norm = lax.rsqrt(jnp.sum(p * p))
    h_ref[...] = h
    sc_ref[...] = jnp.dot(h, p, preferred_element_type=_f32) * inv_norm


def _tc_matmul(aggr, xprev, wrel, b, wroot, p):
    grid = NPAD // RB
    return pl.pallas_call(
        _matmul_body,
        grid=(grid,),
        in_specs=[
            pl.BlockSpec((NC, RB, D), lambda i: (0, i, 0)),
            pl.BlockSpec((RB, D), lambda i: (i, 0)),
            pl.BlockSpec((D, D), lambda i: (0, 0)),
            pl.BlockSpec((1, D), lambda i: (0, 0)),
            pl.BlockSpec((D, D), lambda i: (0, 0)),
            pl.BlockSpec((D, 1), lambda i: (0, 0)),
        ],
        out_specs=[
            pl.BlockSpec((RB, D), lambda i: (i, 0)),
            pl.BlockSpec((RB, 1), lambda i: (i, 0)),
        ],
        out_shape=[jax.ShapeDtypeStruct((NPAD, D), _f32),
                   jax.ShapeDtypeStruct((NPAD, 1), _f32)],
    )(aggr, xprev, wrel, b.reshape(1, D), wroot, p.reshape(D, 1))


def _pool_body(h_ref, sc_ref, scr_ref, alive_ref, alivec_ref,
               f_ref, alive_out_ref, *, k):
    sm = jnp.where(alive_ref[...] > 0.0, scr_ref[...], _f32(-3e38))
    u = _monotone_u32(sm)

    def body(i, pfx):
        cand = pfx | (jnp.uint32(1) << (31 - i))
        cnt = jnp.sum((u >= cand).astype(_i32))
        return jnp.where(cnt >= k, cand, pfx)

    thr = lax.fori_loop(0, 32, body, jnp.uint32(0))
    alive_out_ref[...] = (u >= thr).astype(_f32)
    sc = sc_ref[...]
    alive_col = jnp.where(
        (_monotone_u32(sc) >= thr) & (alivec_ref[...] > 0.0), 1.0, 0.0)
    f_ref[...] = h_ref[...] * jnp.tanh(sc) * alive_col


def _tc_pool(h, sc, sc_r, alive_prev_r, alive_prev_col, k):
    return pl.pallas_call(
        functools.partial(_pool_body, k=k),
        out_shape=[jax.ShapeDtypeStruct((NPAD, D), _f32),
                   jax.ShapeDtypeStruct((SR, D), _f32)],
    )(h, sc, sc_r, alive_prev_r, alive_prev_col)


def _final_body(h_ref, sc_ref, scr_ref, alive_ref, out_ref):
    sm = jnp.where(alive_ref[...] > 0.0, scr_ref[...], _f32(-3e38))
    fiota = (lax.broadcasted_iota(_i32, (SR, D), 0) * D
             + lax.broadcasted_iota(_i32, (SR, D), 1))
    rows = []
    svals = []
    for _ in range(10):
        m = jnp.max(sm)
        idx = jnp.min(jnp.where(sm == m, fiota, NPAD))
        rows.append(h_ref[pl.ds(idx, 1), :])
        svals.append(sc_ref[pl.ds(idx, 1), :])
        sm = jnp.where(fiota == idx, _f32(-3e38), sm)
    v = jnp.concatenate(rows, axis=0) * jnp.tanh(jnp.concatenate(svals, axis=0))
    mx = jnp.max(v, axis=1, keepdims=True)
    v = v - mx
    out_ref[...] = v - jnp.log(jnp.sum(jnp.exp(v), axis=1, keepdims=True))


def _tc_final(h, sc, sc_r, alive_prev_r):
    return pl.pallas_call(
        _final_body,
        out_shape=jax.ShapeDtypeStruct((10, D), _f32),
    )(h, sc, sc_r, alive_prev_r)


def kernel(x, edge_index, batch, W_rel1, b1, W_root1, p1,
           W_rel2, b2, W_root2, p2, W_rel3, b3, W_root3, p3):
    src = edge_index[0].astype(_i32).reshape(NW, EPW)
    dst = edge_index[1].astype(_i32).reshape(NW, NCHUNK, CH)
    xp = jnp.pad(x, ((0, NPAD - N), (0, 0)))
    alive0_r = (jnp.arange(NPAD, dtype=_i32).reshape(SR, D) < N).astype(_f32)
    alive0_c = alive0_r.reshape(NPAD, 1)

    sc_agg = _sc_aggregate()

    agg1 = sc_agg(xp, src, dst)
    h1, s1 = _tc_matmul(agg1, xp, W_rel1, b1, W_root1, p1)
    f1, alive1_r = _tc_pool(h1, s1, s1.reshape(SR, D), alive0_r, alive0_c,
                            8000)

    agg2 = sc_agg(f1, src, dst)
    h2, s2 = _tc_matmul(agg2, f1, W_rel2, b2, W_root2, p2)
    f2, alive2_r = _tc_pool(h2, s2, s2.reshape(SR, D), alive1_r,
                            alive1_r.reshape(NPAD, 1), 6400)

    agg3 = sc_agg(f2, src, dst)
    h3, s3 = _tc_matmul(agg3, f2, W_rel3, b3, W_root3, p3)
    return _tc_final(h3, s3, s3.reshape(SR, D), alive2_r)
